# baseline (speedup 1.0000x reference)
import jax
import jax.numpy as jnp
from jax.experimental import pallas as pl
from jax.experimental.pallas import tpu as pltpu

_N = 10000
_NFEAT = 128
_NHID = 16
_NCLASS = 10
_BLK = 80
_ALPHA = 0.2
_POW = 0.72


def _fused(x_ref, adj1_ref, adj2_ref, w1_ref, w2_ref, b1_ref, b2_ref,
           wa_ref, ct_ref, xo_ref, q_ref, xw1_ref, xw2_ref):
    i = pl.program_id(0)

    @pl.when(i == 0)
    def _():
        xb = x_ref[...].astype(jnp.bfloat16)
        xw1 = jnp.dot(xb, w1_ref[...].astype(jnp.bfloat16),
                      preferred_element_type=jnp.float32)
        xw2 = jnp.dot(xb, w2_ref[...].astype(jnp.bfloat16),
                      preferred_element_type=jnp.float32)
        hi1 = xw1.astype(jnp.bfloat16)
        hi2 = xw2.astype(jnp.bfloat16)
        lo1 = (xw1 - hi1.astype(jnp.float32)).astype(jnp.bfloat16)
        lo2 = (xw2 - hi2.astype(jnp.float32)).astype(jnp.bfloat16)
        xw1_ref[...] = jnp.concatenate([hi1, lo1], axis=1)
        xw2_ref[...] = jnp.concatenate([hi2, lo2], axis=1)

    a1 = adj1_ref[...].astype(jnp.bfloat16)
    a2 = adj2_ref[...].astype(jnp.bfloat16)
    ee1 = jnp.dot(a1, xw1_ref[...], preferred_element_type=jnp.float32)
    ee2 = jnp.dot(a2, xw2_ref[...], preferred_element_type=jnp.float32)
    e1 = ee1[:, :_NHID] + ee1[:, _NHID:] + b1_ref[...]
    e2 = ee2[:, :_NHID] + ee2[:, _NHID:] + b2_ref[...]

    s1 = jnp.sum(e1 * wa_ref[...], axis=1, keepdims=True)
    s2 = jnp.sum(e2 * wa_ref[...], axis=1, keepdims=True)
    m = jnp.maximum(s1, s2)
    p1 = jnp.exp(s1 - m)
    p2 = jnp.exp(s2 - m)
    xo = (p1 * e1 + p2 * e2) / (p1 + p2)
    xo_ref[...] = xo

    ct = ct_ref[...]
    csq = jnp.sum(ct * ct, axis=0, keepdims=True)
    cross = jnp.dot(xo, ct, preferred_element_type=jnp.float32)
    dist = jnp.sum(xo * xo, axis=1, keepdims=True) - 2.0 * cross + csq
    p = 1.0 / (1.0 + dist * (1.0 / _ALPHA))
    qu = jnp.exp(_POW * jnp.log(p))
    q_ref[...] = qu / jnp.sum(qu, axis=1, keepdims=True)


def kernel(x, adj1, adj2, W1, b1, W2, b2, Wa, cluster):
    b1r = b1.reshape(1, _NHID)
    b2r = b2.reshape(1, _NHID)
    war = Wa.reshape(1, _NHID)
    ct = cluster.T

    grid = (_N // _BLK,)
    xo, q = pl.pallas_call(
        _fused,
        grid=grid,
        in_specs=[
            pl.BlockSpec((_N, _NFEAT), lambda i: (0, 0)),
            pl.BlockSpec((_BLK, _N), lambda i: (i, 0)),
            pl.BlockSpec((_BLK, _N), lambda i: (i, 0)),
            pl.BlockSpec((_NFEAT, _NHID), lambda i: (0, 0)),
            pl.BlockSpec((_NFEAT, _NHID), lambda i: (0, 0)),
            pl.BlockSpec((1, _NHID), lambda i: (0, 0)),
            pl.BlockSpec((1, _NHID), lambda i: (0, 0)),
            pl.BlockSpec((1, _NHID), lambda i: (0, 0)),
            pl.BlockSpec((_NHID, _NCLASS), lambda i: (0, 0)),
        ],
        out_specs=[
            pl.BlockSpec((_BLK, _NHID), lambda i: (i, 0)),
            pl.BlockSpec((_BLK, _NCLASS), lambda i: (i, 0)),
        ],
        out_shape=[
            jax.ShapeDtypeStruct((_N, _NHID), jnp.float32),
            jax.ShapeDtypeStruct((_N, _NCLASS), jnp.float32),
        ],
        scratch_shapes=[
            pltpu.VMEM((_N, 2 * _NHID), jnp.bfloat16),
            pltpu.VMEM((_N, 2 * _NHID), jnp.bfloat16),
        ],
    )(x, adj1, adj2, W1, W2, b1r, b2r, war, ct)
    return (xo, q)


# speedup vs baseline: 1.1565x; 1.1565x over previous
import jax
import jax.numpy as jnp
from jax.experimental import pallas as pl
from jax.experimental.pallas import tpu as pltpu

_N = 10000
_NFEAT = 128
_NHID = 16
_NCLASS = 10
_BLK = 200
_ALPHA = 0.2
_POW = 0.72


def _fused(x_ref, adj1_ref, adj2_ref, w1_ref, w2_ref, b1_ref, b2_ref,
           wa_ref, ct_ref, xo_ref, q_ref, xw1_ref, xw2_ref):
    i = pl.program_id(0)

    @pl.when(i == 0)
    def _():
        xb = x_ref[...].astype(jnp.bfloat16)
        xw1 = jnp.dot(xb, w1_ref[...].astype(jnp.bfloat16),
                      preferred_element_type=jnp.float32)
        xw2 = jnp.dot(xb, w2_ref[...].astype(jnp.bfloat16),
                      preferred_element_type=jnp.float32)
        hi1 = xw1.astype(jnp.bfloat16)
        hi2 = xw2.astype(jnp.bfloat16)
        lo1 = (xw1 - hi1.astype(jnp.float32)).astype(jnp.bfloat16)
        lo2 = (xw2 - hi2.astype(jnp.float32)).astype(jnp.bfloat16)
        xw1_ref[...] = jnp.concatenate([hi1, lo1], axis=1)
        xw2_ref[...] = jnp.concatenate([hi2, lo2], axis=1)

    a1 = adj1_ref[...].astype(jnp.bfloat16)
    a2 = adj2_ref[...].astype(jnp.bfloat16)
    ee1 = jnp.dot(a1, xw1_ref[...], preferred_element_type=jnp.float32)
    ee2 = jnp.dot(a2, xw2_ref[...], preferred_element_type=jnp.float32)
    e1 = ee1[:, :_NHID] + ee1[:, _NHID:] + b1_ref[...]
    e2 = ee2[:, :_NHID] + ee2[:, _NHID:] + b2_ref[...]

    s1 = jnp.sum(e1 * wa_ref[...], axis=1, keepdims=True)
    s2 = jnp.sum(e2 * wa_ref[...], axis=1, keepdims=True)
    m = jnp.maximum(s1, s2)
    p1 = jnp.exp(s1 - m)
    p2 = jnp.exp(s2 - m)
    xo = (p1 * e1 + p2 * e2) / (p1 + p2)
    xo_ref[pl.ds(i * _BLK, _BLK), :] = xo

    ct = ct_ref[...]
    csq = jnp.sum(ct * ct, axis=0, keepdims=True)
    cross = jnp.dot(xo, ct, preferred_element_type=jnp.float32)
    dist = jnp.sum(xo * xo, axis=1, keepdims=True) - 2.0 * cross + csq
    p = 1.0 / (1.0 + dist * (1.0 / _ALPHA))
    qu = jnp.exp(_POW * jnp.log(p))
    q_ref[pl.ds(i * _BLK, _BLK), :] = qu / jnp.sum(qu, axis=1, keepdims=True)


def kernel(x, adj1, adj2, W1, b1, W2, b2, Wa, cluster):
    b1r = b1.reshape(1, _NHID)
    b2r = b2.reshape(1, _NHID)
    war = Wa.reshape(1, _NHID)
    ct = cluster.T

    grid = (_N // _BLK,)
    xo, q = pl.pallas_call(
        _fused,
        grid=grid,
        in_specs=[
            pl.BlockSpec((_N, _NFEAT), lambda i: (0, 0)),
            pl.BlockSpec((_BLK, _N), lambda i: (i, 0)),
            pl.BlockSpec((_BLK, _N), lambda i: (i, 0)),
            pl.BlockSpec((_NFEAT, _NHID), lambda i: (0, 0)),
            pl.BlockSpec((_NFEAT, _NHID), lambda i: (0, 0)),
            pl.BlockSpec((1, _NHID), lambda i: (0, 0)),
            pl.BlockSpec((1, _NHID), lambda i: (0, 0)),
            pl.BlockSpec((1, _NHID), lambda i: (0, 0)),
            pl.BlockSpec((_NHID, _NCLASS), lambda i: (0, 0)),
        ],
        out_specs=[
            pl.BlockSpec((_N, _NHID), lambda i: (0, 0)),
            pl.BlockSpec((_N, _NCLASS), lambda i: (0, 0)),
        ],
        out_shape=[
            jax.ShapeDtypeStruct((_N, _NHID), jnp.float32),
            jax.ShapeDtypeStruct((_N, _NCLASS), jnp.float32),
        ],
        scratch_shapes=[
            pltpu.VMEM((_N, 2 * _NHID), jnp.bfloat16),
            pltpu.VMEM((_N, 2 * _NHID), jnp.bfloat16),
        ],
    )(x, adj1, adj2, W1, W2, b1r, b2r, war, ct)
    return (xo, q)


# speedup vs baseline: 1.1668x; 1.0088x over previous
import jax
import jax.numpy as jnp
from jax.experimental import pallas as pl
from jax.experimental.pallas import tpu as pltpu

_N = 10000
_NFEAT = 128
_NHID = 16
_NCLASS = 10
_BLK = 200
_ALPHA = 0.2
_POW = 0.72


def _fused(x_ref, adj1_ref, adj2_ref, w1_ref, w2_ref, b1_ref, b2_ref,
           wa_ref, ct_ref, xo_ref, q_ref, xw1_ref, xw2_ref):
    i = pl.program_id(0)

    @pl.when(i == 0)
    def _():
        xb = x_ref[...].astype(jnp.bfloat16)
        xw1 = jnp.dot(xb, w1_ref[...].astype(jnp.bfloat16),
                      preferred_element_type=jnp.float32)
        xw2 = jnp.dot(xb, w2_ref[...].astype(jnp.bfloat16),
                      preferred_element_type=jnp.float32)
        hi1 = xw1.astype(jnp.bfloat16)
        hi2 = xw2.astype(jnp.bfloat16)
        lo1 = (xw1 - hi1.astype(jnp.float32)).astype(jnp.bfloat16)
        lo2 = (xw2 - hi2.astype(jnp.float32)).astype(jnp.bfloat16)
        xw1_ref[...] = jnp.concatenate([hi1, lo1], axis=1)
        xw2_ref[...] = jnp.concatenate([hi2, lo2], axis=1)

    a1 = adj1_ref[...].astype(jnp.bfloat16)
    a2 = adj2_ref[...].astype(jnp.bfloat16)
    ee1 = jnp.dot(a1, xw1_ref[...], preferred_element_type=jnp.float32)
    ee2 = jnp.dot(a2, xw2_ref[...], preferred_element_type=jnp.float32)
    e1 = ee1[:, :_NHID] + ee1[:, _NHID:] + b1_ref[...]
    e2 = ee2[:, :_NHID] + ee2[:, _NHID:] + b2_ref[...]

    s1 = jnp.sum(e1 * wa_ref[...], axis=1, keepdims=True)
    s2 = jnp.sum(e2 * wa_ref[...], axis=1, keepdims=True)
    m = jnp.maximum(s1, s2)
    p1 = jnp.exp(s1 - m)
    p2 = jnp.exp(s2 - m)
    xo = (p1 * e1 + p2 * e2) / (p1 + p2)
    xo_ref[...] = xo

    ct = ct_ref[...]
    csq = jnp.sum(ct * ct, axis=0, keepdims=True)
    cross = jnp.dot(xo, ct, preferred_element_type=jnp.float32)
    dist = jnp.sum(xo * xo, axis=1, keepdims=True) - 2.0 * cross + csq
    p = 1.0 / (1.0 + dist * (1.0 / _ALPHA))
    qu = jnp.exp(_POW * jnp.log(p))
    q_ref[...] = qu / jnp.sum(qu, axis=1, keepdims=True)


def kernel(x, adj1, adj2, W1, b1, W2, b2, Wa, cluster):
    b1r = b1.reshape(1, _NHID)
    b2r = b2.reshape(1, _NHID)
    war = Wa.reshape(1, _NHID)
    ct = cluster.T

    grid = (_N // _BLK,)
    xo, q = pl.pallas_call(
        _fused,
        grid=grid,
        in_specs=[
            pl.BlockSpec((_N, _NFEAT), lambda i: (0, 0)),
            pl.BlockSpec((_BLK, _N), lambda i: (i, 0)),
            pl.BlockSpec((_BLK, _N), lambda i: (i, 0)),
            pl.BlockSpec((_NFEAT, _NHID), lambda i: (0, 0)),
            pl.BlockSpec((_NFEAT, _NHID), lambda i: (0, 0)),
            pl.BlockSpec((1, _NHID), lambda i: (0, 0)),
            pl.BlockSpec((1, _NHID), lambda i: (0, 0)),
            pl.BlockSpec((1, _NHID), lambda i: (0, 0)),
            pl.BlockSpec((_NHID, _NCLASS), lambda i: (0, 0)),
        ],
        out_specs=[
            pl.BlockSpec((_BLK, _NHID), lambda i: (i, 0)),
            pl.BlockSpec((_BLK, _NCLASS), lambda i: (i, 0)),
        ],
        out_shape=[
            jax.ShapeDtypeStruct((_N, _NHID), jnp.float32),
            jax.ShapeDtypeStruct((_N, _NCLASS), jnp.float32),
        ],
        scratch_shapes=[
            pltpu.VMEM((_N, 2 * _NHID), jnp.bfloat16),
            pltpu.VMEM((_N, 2 * _NHID), jnp.bfloat16),
        ],
    )(x, adj1, adj2, W1, W2, b1r, b2r, war, ct)
    return (xo, q)


# speedup vs baseline: 1.1764x; 1.0083x over previous
import jax
import jax.numpy as jnp
from jax.experimental import pallas as pl
from jax.experimental.pallas import tpu as pltpu

_N = 10000
_NFEAT = 128
_NHID = 16
_NCLASS = 10
_BLK = 200
_NBLK = _N // _BLK
_DEPTH = 3
_ALPHA = 0.2
_POW = 0.72


def _fused(x_ref, adj1_ref, adj2_ref, w1_ref, w2_ref, b1_ref, b2_ref,
           wa_ref, ct_ref, xo_ref, q_ref, xw1_ref, xw2_ref,
           buf1, buf2, sem1, sem2):
    i = pl.program_id(0)

    def issue(step, slot):
        pltpu.make_async_copy(
            adj1_ref.at[pl.ds(step * _BLK, _BLK), :], buf1.at[slot],
            sem1.at[slot]).start()
        pltpu.make_async_copy(
            adj2_ref.at[pl.ds(step * _BLK, _BLK), :], buf2.at[slot],
            sem2.at[slot]).start()

    @pl.when(i == 0)
    def _():
        for p in range(_DEPTH):
            issue(p, p)
        xb = x_ref[...].astype(jnp.bfloat16)
        xw1 = jnp.dot(xb, w1_ref[...].astype(jnp.bfloat16),
                      preferred_element_type=jnp.float32)
        xw2 = jnp.dot(xb, w2_ref[...].astype(jnp.bfloat16),
                      preferred_element_type=jnp.float32)
        hi1 = xw1.astype(jnp.bfloat16)
        hi2 = xw2.astype(jnp.bfloat16)
        lo1 = (xw1 - hi1.astype(jnp.float32)).astype(jnp.bfloat16)
        lo2 = (xw2 - hi2.astype(jnp.float32)).astype(jnp.bfloat16)
        xw1_ref[...] = jnp.concatenate([hi1, lo1], axis=1)
        xw2_ref[...] = jnp.concatenate([hi2, lo2], axis=1)

    slot = jax.lax.rem(i, _DEPTH)
    pltpu.make_async_copy(
        adj1_ref.at[pl.ds(0, _BLK), :], buf1.at[slot], sem1.at[slot]).wait()
    pltpu.make_async_copy(
        adj2_ref.at[pl.ds(0, _BLK), :], buf2.at[slot], sem2.at[slot]).wait()

    a1 = buf1[slot].astype(jnp.bfloat16)
    a2 = buf2[slot].astype(jnp.bfloat16)
    ee1 = jnp.dot(a1, xw1_ref[...], preferred_element_type=jnp.float32)
    ee2 = jnp.dot(a2, xw2_ref[...], preferred_element_type=jnp.float32)
    e1 = ee1[:, :_NHID] + ee1[:, _NHID:] + b1_ref[...]
    e2 = ee2[:, :_NHID] + ee2[:, _NHID:] + b2_ref[...]

    @pl.when(i + _DEPTH < _NBLK)
    def _():
        issue(i + _DEPTH, slot)

    s1 = jnp.sum(e1 * wa_ref[...], axis=1, keepdims=True)
    s2 = jnp.sum(e2 * wa_ref[...], axis=1, keepdims=True)
    m = jnp.maximum(s1, s2)
    p1 = jnp.exp(s1 - m)
    p2 = jnp.exp(s2 - m)
    xo = (p1 * e1 + p2 * e2) / (p1 + p2)
    xo_ref[...] = xo

    ct = ct_ref[...]
    csq = jnp.sum(ct * ct, axis=0, keepdims=True)
    cross = jnp.dot(xo, ct, preferred_element_type=jnp.float32)
    dist = jnp.sum(xo * xo, axis=1, keepdims=True) - 2.0 * cross + csq
    p = 1.0 / (1.0 + dist * (1.0 / _ALPHA))
    qu = jnp.exp(_POW * jnp.log(p))
    q_ref[...] = qu / jnp.sum(qu, axis=1, keepdims=True)


def kernel(x, adj1, adj2, W1, b1, W2, b2, Wa, cluster):
    b1r = b1.reshape(1, _NHID)
    b2r = b2.reshape(1, _NHID)
    war = Wa.reshape(1, _NHID)
    ct = cluster.T

    grid = (_NBLK,)
    xo, q = pl.pallas_call(
        _fused,
        grid=grid,
        in_specs=[
            pl.BlockSpec((_N, _NFEAT), lambda i: (0, 0)),
            pl.BlockSpec(memory_space=pl.ANY),
            pl.BlockSpec(memory_space=pl.ANY),
            pl.BlockSpec((_NFEAT, _NHID), lambda i: (0, 0)),
            pl.BlockSpec((_NFEAT, _NHID), lambda i: (0, 0)),
            pl.BlockSpec((1, _NHID), lambda i: (0, 0)),
            pl.BlockSpec((1, _NHID), lambda i: (0, 0)),
            pl.BlockSpec((1, _NHID), lambda i: (0, 0)),
            pl.BlockSpec((_NHID, _NCLASS), lambda i: (0, 0)),
        ],
        out_specs=[
            pl.BlockSpec((_BLK, _NHID), lambda i: (i, 0)),
            pl.BlockSpec((_BLK, _NCLASS), lambda i: (i, 0)),
        ],
        out_shape=[
            jax.ShapeDtypeStruct((_N, _NHID), jnp.float32),
            jax.ShapeDtypeStruct((_N, _NCLASS), jnp.float32),
        ],
        scratch_shapes=[
            pltpu.VMEM((_N, 2 * _NHID), jnp.bfloat16),
            pltpu.VMEM((_N, 2 * _NHID), jnp.bfloat16),
            pltpu.VMEM((_DEPTH, _BLK, _N), jnp.float32),
            pltpu.VMEM((_DEPTH, _BLK, _N), jnp.float32),
            pltpu.SemaphoreType.DMA((_DEPTH,)),
            pltpu.SemaphoreType.DMA((_DEPTH,)),
        ],
    )(x, adj1, adj2, W1, W2, b1r, b2r, war, ct)
    return (xo, q)
